# baseline (speedup 1.0000x reference)
import functools
import math

import jax
import jax.numpy as jnp
from jax import lax
from jax.experimental import pallas as pl
from jax.experimental.pallas import tpu as pltpu
from jax.experimental.pallas import tpu_sc as plsc

N = 4096
C = 512
DEPTH = 3

_NW = 32
_SPMEM_BYTES = 192 * 1024


@functools.cache
def _make_sc_gather(V, D, B):
    b_per_w = B // _NW
    assert b_per_w * _NW == B and b_per_w % 8 == 0
    lim = max(8, _SPMEM_BYTES // (D * 4))
    ch = 8
    while ch * 2 <= min(b_per_w, lim):
        ch *= 2
    n_chunks = b_per_w // ch
    assert n_chunks * ch == b_per_w
    mesh = plsc.VectorSubcoreMesh(core_axis_name="c", subcore_axis_name="s")

    @functools.partial(
        pl.kernel, mesh=mesh,
        out_type=jax.ShapeDtypeStruct((B, D), jnp.float32),
        scratch_types=[
            pltpu.VMEM((b_per_w,), jnp.int32),
            pltpu.VMEM((ch, D), jnp.float32),
            pltpu.VMEM((ch, D), jnp.float32),
            pltpu.SemaphoreType.DMA,
            pltpu.SemaphoreType.DMA,
        ],
    )
    def gk(table_hbm, idx_hbm, out_hbm, idx_v, rows0_v, rows1_v, sem0, sem1):
        wid = lax.axis_index("s") * 2 + lax.axis_index("c")
        base = wid * b_per_w
        pltpu.sync_copy(idx_hbm.at[pl.ds(base, b_per_w)], idx_v)
        bufs = (rows0_v, rows1_v)
        sems = (sem0, sem1)
        if n_chunks == 1:
            pltpu.async_copy(table_hbm.at[idx_v], rows0_v, sem0).wait()
            pltpu.sync_copy(rows0_v, out_hbm.at[pl.ds(base, ch)])
            return
        pend = [None, None]
        pend[0] = pltpu.async_copy(
            table_hbm.at[idx_v.at[pl.ds(0, ch)]], bufs[0], sems[0])
        for c in range(1, n_chunks):
            pend[c % 2] = pltpu.async_copy(
                table_hbm.at[idx_v.at[pl.ds(c * ch, ch)]], bufs[c % 2],
                sems[c % 2])
            pend[(c - 1) % 2].wait()
            pltpu.sync_copy(bufs[(c - 1) % 2],
                            out_hbm.at[pl.ds(base + (c - 1) * ch, ch)])
        pend[(n_chunks - 1) % 2].wait()
        pltpu.sync_copy(bufs[(n_chunks - 1) % 2],
                        out_hbm.at[pl.ds(base + (n_chunks - 1) * ch, ch)])

    return gk


def _sc_gather(table, idx):
    V, D = table.shape
    (B,) = idx.shape
    return _make_sc_gather(V, D, B)(table, idx)


@functools.cache
def _make_sc_gather2(V1, D1, V2, D2, B):
    b_per_w = B // _NW
    assert b_per_w * _NW == B and b_per_w % 8 == 0
    budget = 96 * 1024

    def _ch(D):
        lim = max(8, budget // (D * 4))
        c = 8
        while c * 2 <= min(b_per_w, lim):
            c *= 2
        return c

    ch1, ch2 = _ch(D1), _ch(D2)
    n1, n2 = b_per_w // ch1, b_per_w // ch2
    mesh = plsc.VectorSubcoreMesh(core_axis_name="c", subcore_axis_name="s")

    @functools.partial(
        pl.kernel, mesh=mesh,
        out_type=[jax.ShapeDtypeStruct((B, D1), jnp.float32),
                  jax.ShapeDtypeStruct((B, D2), jnp.float32)],
        scratch_types=[
            pltpu.VMEM((b_per_w,), jnp.int32),
            pltpu.VMEM((ch1, D1), jnp.float32),
            pltpu.VMEM((ch1, D1), jnp.float32),
            pltpu.VMEM((ch2, D2), jnp.float32),
            pltpu.VMEM((ch2, D2), jnp.float32),
            pltpu.SemaphoreType.DMA,
            pltpu.SemaphoreType.DMA,
        ],
    )
    def gk(t1_hbm, t2_hbm, idx_hbm, o1_hbm, o2_hbm,
           idx_v, a0, a1, b0, b1, sem0, sem1):
        wid = lax.axis_index("s") * 2 + lax.axis_index("c")
        base = wid * b_per_w
        pltpu.sync_copy(idx_hbm.at[pl.ds(base, b_per_w)], idx_v)
        for table, out, bufs, ch, n in ((t1_hbm, o1_hbm, (a0, a1), ch1, n1),
                                        (t2_hbm, o2_hbm, (b0, b1), ch2, n2)):
            sems = (sem0, sem1)
            pend = [None, None]
            pend[0] = pltpu.async_copy(
                table.at[idx_v.at[pl.ds(0, ch)]], bufs[0], sems[0])
            for c in range(1, n):
                pend[c % 2] = pltpu.async_copy(
                    table.at[idx_v.at[pl.ds(c * ch, ch)]], bufs[c % 2],
                    sems[c % 2])
                pend[(c - 1) % 2].wait()
                pltpu.sync_copy(bufs[(c - 1) % 2],
                                out.at[pl.ds(base + (c - 1) * ch, ch)])
            pend[(n - 1) % 2].wait()
            pltpu.sync_copy(bufs[(n - 1) % 2],
                            out.at[pl.ds(base + (n - 1) * ch, ch)])

    return gk


def _sc_gather2(t1, t2, idx):
    return _make_sc_gather2(t1.shape[0], t1.shape[1],
                            t2.shape[0], t2.shape[1], idx.shape[0])(t1, t2, idx)


def _pool_kernel(x_ref, p_ref, xs_ref, perm_ref, pos_ref, msk_ref, *, K, k):
    p = p_ref[...]
    pn = jnp.sqrt(jnp.sum(p * p))
    x = x_ref[...]
    s_col = jnp.tanh(
        lax.dot_general(x, p, (((1,), (1,)), ((), ())),
                        preferred_element_type=jnp.float32) / pn)
    xs_ref[...] = x * s_col
    s_row = jnp.transpose(s_col)
    idx_row = lax.broadcasted_iota(jnp.int32, (1, K), 1)
    rank = jnp.zeros((1, K), jnp.float32)
    CH = 512
    for c in range(K // CH):
        sj = lax.slice(s_col, (c * CH, 0), ((c + 1) * CH, 1))
        ij = lax.broadcasted_iota(jnp.int32, (CH, 1), 0) + c * CH
        gt = (sj > s_row).astype(jnp.float32)
        eq = ((sj == s_row) & (ij < idx_row)).astype(jnp.float32)
        rank = rank + jnp.sum(gt + eq, axis=0, keepdims=True)
    mask_row = (rank < float(k)).astype(jnp.float32)
    pos_parts = []
    for c in range(K // CH):
        jcol = lax.broadcasted_iota(jnp.int32, (K, 1), 0)
        irow = lax.broadcasted_iota(jnp.int32, (1, CH), 1) + c * CH
        tri = (jcol < irow).astype(jnp.float32)
        pos_parts.append(jnp.dot(mask_row, tri,
                                 preferred_element_type=jnp.float32))
    pos_row = jnp.concatenate(pos_parts, axis=1)
    fidx_row = idx_row.astype(jnp.float32)
    RCH = min(k, 512)
    perm_parts = []
    for rc in range(k // RCH):
        r_col = lax.broadcasted_iota(jnp.int32, (RCH, 1), 0) + rc * RCH
        hit = ((pos_row == r_col.astype(jnp.float32)) &
               (mask_row > 0)).astype(jnp.float32)
        perm_parts.append(jnp.sum(hit * fidx_row, axis=1, keepdims=True))
    perm_col = jnp.concatenate(perm_parts, axis=0)
    perm_ref[...] = perm_col.astype(jnp.int32)
    pos_col = jnp.transpose(jnp.minimum(pos_row, float(k - 1)))
    pos_ref[...] = pos_col.astype(jnp.int32)
    msk_ref[...] = jnp.transpose(mask_row)


def _pool(x, p):
    K = x.shape[0]
    k = K // 2
    return pl.pallas_call(
        functools.partial(_pool_kernel, K=K, k=k),
        out_shape=[jax.ShapeDtypeStruct((K, C), jnp.float32),
                   jax.ShapeDtypeStruct((k, 1), jnp.int32),
                   jax.ShapeDtypeStruct((K, 1), jnp.int32),
                   jax.ShapeDtypeStruct((K, 1), jnp.float32)],
    )(x, p)


_BT = 512


def _tr_kernel(x_ref, o_ref):
    o_ref[...] = jnp.transpose(x_ref[...])


def _transpose(x):
    R, Q = x.shape
    return pl.pallas_call(
        _tr_kernel,
        grid=(R // _BT, Q // _BT),
        in_specs=[pl.BlockSpec((_BT, _BT), lambda i, j: (i, j))],
        out_specs=pl.BlockSpec((_BT, _BT), lambda i, j: (j, i)),
        out_shape=jax.ShapeDtypeStruct((Q, R), jnp.float32),
    )(x)


def _xw_kernel(x_ref, w_ref, b_ref, o_ref):
    o_ref[...] = (jnp.dot(x_ref[...], w_ref[...],
                          preferred_element_type=jnp.float32) + b_ref[...])


def _xw(x, w, b):
    M = x.shape[0]
    bm = min(M, 1024)
    return pl.pallas_call(
        _xw_kernel,
        grid=(M // bm,),
        in_specs=[
            pl.BlockSpec((bm, C), lambda i: (i, 0)),
            pl.BlockSpec((C, C), lambda i: (0, 0)),
            pl.BlockSpec((1, C), lambda i: (0, 0)),
        ],
        out_specs=pl.BlockSpec((bm, C), lambda i: (i, 0)),
        out_shape=jax.ShapeDtypeStruct((M, C), jnp.float32),
    )(x, w, b)


def _xw_sum_kernel(xa_ref, xb_ref, m_ref, w_ref, b_ref, o_ref):
    xs = xa_ref[...] + xb_ref[...] * m_ref[...]
    o_ref[...] = (jnp.dot(xs, w_ref[...],
                          preferred_element_type=jnp.float32) + b_ref[...])


def _xw_sum(xa, xb, m_col, w, b):
    M = xa.shape[0]
    bm = min(M, 1024)
    return pl.pallas_call(
        _xw_sum_kernel,
        grid=(M // bm,),
        in_specs=[
            pl.BlockSpec((bm, C), lambda i: (i, 0)),
            pl.BlockSpec((bm, C), lambda i: (i, 0)),
            pl.BlockSpec((bm, 1), lambda i: (i, 0)),
            pl.BlockSpec((C, C), lambda i: (0, 0)),
            pl.BlockSpec((1, C), lambda i: (0, 0)),
        ],
        out_specs=pl.BlockSpec((bm, C), lambda i: (i, 0)),
        out_shape=jax.ShapeDtypeStruct((M, C), jnp.float32),
    )(xa, xb, m_col, w, b)


def _hmm_kernel(h_ref, t_ref, o_ref):
    o_ref[...] = jnp.maximum(
        jnp.dot(h_ref[...], t_ref[...], preferred_element_type=jnp.float32), 0.0)


def _hmm_t_kernel(h_ref, t_ref, o_ref):
    acc = lax.dot_general(h_ref[...], t_ref[...], (((0,), (0,)), ((), ())),
                          preferred_element_type=jnp.float32)
    o_ref[...] = jnp.maximum(acc, 0.0)


def _hconv(S, t, transposed):
    M = t.shape[0]
    bm = min(M, 512)
    if transposed:
        return pl.pallas_call(
            _hmm_t_kernel,
            grid=(M // bm,),
            in_specs=[
                pl.BlockSpec((M, bm), lambda i: (0, i)),
                pl.BlockSpec((M, C), lambda i: (0, 0)),
            ],
            out_specs=pl.BlockSpec((bm, C), lambda i: (i, 0)),
            out_shape=jax.ShapeDtypeStruct((M, C), jnp.float32),
        )(S, t)
    return pl.pallas_call(
        _hmm_kernel,
        grid=(M // bm,),
        in_specs=[
            pl.BlockSpec((bm, M), lambda i: (i, 0)),
            pl.BlockSpec((M, C), lambda i: (0, 0)),
        ],
        out_specs=pl.BlockSpec((bm, C), lambda i: (i, 0)),
        out_shape=jax.ShapeDtypeStruct((M, C), jnp.float32),
    )(S, t)


def _gather_rows(table, idx_2d):
    return _sc_gather(table, idx_2d.reshape(-1))


def kernel(feat, H, pool_w0, pool_w1, pool_w2, Wd0, bd0, Wd1, bd1, Wd2, bd2,
           Wu0, bu0, Wu1, bu1, Wu2, bu2):
    pool_ws = [pool_w0, pool_w1, pool_w2]
    Wds = [Wd0, Wd1, Wd2]
    bds = [bd0.reshape(1, C), bd1.reshape(1, C), bd2.reshape(1, C)]
    Wus = [Wu0, Wu1, Wu2]
    bus = [bu0.reshape(1, C), bu1.reshape(1, C), bu2.reshape(1, C)]

    x = feat
    S = H
    S_T = False
    xsaved = [feat]
    saved = [(H, False)]
    poss, msks = [], []
    for i in range(DEPTH):
        xs, perm, pos, msk = _pool(x, pool_ws[i])
        poss.append(pos)
        msks.append(msk)
        A, xg = _sc_gather2(S, xs, perm.reshape(-1))
        At = _transpose(A)
        S = _gather_rows(At, perm)
        S_T = not S_T
        t = _xw(xg, Wds[i], bds[i])
        x = _hconv(S, t, S_T)
        if i < DEPTH - 1:
            xsaved.append(x)
            saved.append((S, S_T))
    for i in range(DEPTH):
        j = DEPTH - i - 1
        Sj, Sj_T = saved[j]
        up = _gather_rows(x, poss[j])
        t = _xw_sum(xsaved[j], up, msks[j], Wus[i], bus[i])
        x = _hconv(Sj, t, Sj_T)
    return x


# speedup vs baseline: 1.0099x; 1.0099x over previous
import functools
import math

import jax
import jax.numpy as jnp
from jax import lax
from jax.experimental import pallas as pl
from jax.experimental.pallas import tpu as pltpu
from jax.experimental.pallas import tpu_sc as plsc

N = 4096
C = 512
DEPTH = 3

_NW = 32
_SPMEM_BYTES = 192 * 1024


@functools.cache
def _make_sc_gather(V, D, B):
    b_per_w = B // _NW
    assert b_per_w * _NW == B and b_per_w % 8 == 0
    lim = max(8, _SPMEM_BYTES // (D * 4))
    ch = 8
    while ch * 2 <= min(b_per_w, lim):
        ch *= 2
    n_chunks = b_per_w // ch
    assert n_chunks * ch == b_per_w
    mesh = plsc.VectorSubcoreMesh(core_axis_name="c", subcore_axis_name="s")

    @functools.partial(
        pl.kernel, mesh=mesh,
        out_type=jax.ShapeDtypeStruct((B, D), jnp.float32),
        scratch_types=[
            pltpu.VMEM((b_per_w,), jnp.int32),
            pltpu.VMEM((ch, D), jnp.float32),
            pltpu.VMEM((ch, D), jnp.float32),
            pltpu.SemaphoreType.DMA,
            pltpu.SemaphoreType.DMA,
        ],
    )
    def gk(table_hbm, idx_hbm, out_hbm, idx_v, rows0_v, rows1_v, sem0, sem1):
        wid = lax.axis_index("s") * 2 + lax.axis_index("c")
        base = wid * b_per_w
        pltpu.sync_copy(idx_hbm.at[pl.ds(base, b_per_w)], idx_v)
        bufs = (rows0_v, rows1_v)
        sems = (sem0, sem1)
        if n_chunks == 1:
            pltpu.async_copy(table_hbm.at[idx_v], rows0_v, sem0).wait()
            pltpu.sync_copy(rows0_v, out_hbm.at[pl.ds(base, ch)])
            return
        pend = [None, None]
        pend[0] = pltpu.async_copy(
            table_hbm.at[idx_v.at[pl.ds(0, ch)]], bufs[0], sems[0])
        for c in range(1, n_chunks):
            pend[c % 2] = pltpu.async_copy(
                table_hbm.at[idx_v.at[pl.ds(c * ch, ch)]], bufs[c % 2],
                sems[c % 2])
            pend[(c - 1) % 2].wait()
            pltpu.sync_copy(bufs[(c - 1) % 2],
                            out_hbm.at[pl.ds(base + (c - 1) * ch, ch)])
        pend[(n_chunks - 1) % 2].wait()
        pltpu.sync_copy(bufs[(n_chunks - 1) % 2],
                        out_hbm.at[pl.ds(base + (n_chunks - 1) * ch, ch)])

    return gk


def _sc_gather(table, idx):
    V, D = table.shape
    (B,) = idx.shape
    return _make_sc_gather(V, D, B)(table, idx)


@functools.cache
def _make_sc_gather2(V1, D1, V2, D2, B):
    b_per_w = B // _NW
    assert b_per_w * _NW == B and b_per_w % 8 == 0
    budget = 96 * 1024

    def _ch(D):
        lim = max(8, budget // (D * 4))
        c = 8
        while c * 2 <= min(b_per_w, lim):
            c *= 2
        return c

    ch1, ch2 = _ch(D1), _ch(D2)
    n1, n2 = b_per_w // ch1, b_per_w // ch2
    mesh = plsc.VectorSubcoreMesh(core_axis_name="c", subcore_axis_name="s")

    @functools.partial(
        pl.kernel, mesh=mesh,
        out_type=[jax.ShapeDtypeStruct((B, D1), jnp.float32),
                  jax.ShapeDtypeStruct((B, D2), jnp.float32)],
        scratch_types=[
            pltpu.VMEM((b_per_w,), jnp.int32),
            pltpu.VMEM((ch1, D1), jnp.float32),
            pltpu.VMEM((ch1, D1), jnp.float32),
            pltpu.VMEM((ch2, D2), jnp.float32),
            pltpu.VMEM((ch2, D2), jnp.float32),
            pltpu.SemaphoreType.DMA,
            pltpu.SemaphoreType.DMA,
        ],
    )
    def gk(t1_hbm, t2_hbm, idx_hbm, o1_hbm, o2_hbm,
           idx_v, a0, a1, b0, b1, sem0, sem1):
        wid = lax.axis_index("s") * 2 + lax.axis_index("c")
        base = wid * b_per_w
        pltpu.sync_copy(idx_hbm.at[pl.ds(base, b_per_w)], idx_v)
        for table, out, bufs, ch, n in ((t1_hbm, o1_hbm, (a0, a1), ch1, n1),
                                        (t2_hbm, o2_hbm, (b0, b1), ch2, n2)):
            sems = (sem0, sem1)
            pend = [None, None]
            pend[0] = pltpu.async_copy(
                table.at[idx_v.at[pl.ds(0, ch)]], bufs[0], sems[0])
            for c in range(1, n):
                pend[c % 2] = pltpu.async_copy(
                    table.at[idx_v.at[pl.ds(c * ch, ch)]], bufs[c % 2],
                    sems[c % 2])
                pend[(c - 1) % 2].wait()
                pltpu.sync_copy(bufs[(c - 1) % 2],
                                out.at[pl.ds(base + (c - 1) * ch, ch)])
            pend[(n - 1) % 2].wait()
            pltpu.sync_copy(bufs[(n - 1) % 2],
                            out.at[pl.ds(base + (n - 1) * ch, ch)])

    return gk


def _sc_gather2(t1, t2, idx):
    return _make_sc_gather2(t1.shape[0], t1.shape[1],
                            t2.shape[0], t2.shape[1], idx.shape[0])(t1, t2, idx)


def _pool_kernel(x_ref, p_ref, xs_ref, perm_ref, pos_ref, msk_ref, *, K, k):
    p = p_ref[...]
    pn = jnp.sqrt(jnp.sum(p * p))
    x = x_ref[...]
    s_col = jnp.tanh(
        lax.dot_general(x, p, (((1,), (1,)), ((), ())),
                        preferred_element_type=jnp.float32) / pn)
    xs_ref[...] = x * s_col
    s_row = jnp.transpose(s_col)
    idx_row = lax.broadcasted_iota(jnp.int32, (1, K), 1)
    rank = jnp.zeros((1, K), jnp.float32)
    CH = 512
    for c in range(K // CH):
        sj = lax.slice(s_col, (c * CH, 0), ((c + 1) * CH, 1))
        ij = lax.broadcasted_iota(jnp.int32, (CH, 1), 0) + c * CH
        gt = (sj > s_row).astype(jnp.float32)
        eq = ((sj == s_row) & (ij < idx_row)).astype(jnp.float32)
        rank = rank + jnp.sum(gt + eq, axis=0, keepdims=True)
    mask_row = (rank < float(k)).astype(jnp.float32)
    pos_parts = []
    for c in range(K // CH):
        jcol = lax.broadcasted_iota(jnp.int32, (K, 1), 0)
        irow = lax.broadcasted_iota(jnp.int32, (1, CH), 1) + c * CH
        tri = (jcol < irow).astype(jnp.float32)
        pos_parts.append(jnp.dot(mask_row, tri,
                                 preferred_element_type=jnp.float32))
    pos_row = jnp.concatenate(pos_parts, axis=1)
    fidx_row = idx_row.astype(jnp.float32)
    RCH = min(k, 512)
    perm_parts = []
    for rc in range(k // RCH):
        r_col = lax.broadcasted_iota(jnp.int32, (RCH, 1), 0) + rc * RCH
        hit = ((pos_row == r_col.astype(jnp.float32)) &
               (mask_row > 0)).astype(jnp.float32)
        perm_parts.append(jnp.sum(hit * fidx_row, axis=1, keepdims=True))
    perm_col = jnp.concatenate(perm_parts, axis=0)
    perm_ref[...] = perm_col.astype(jnp.int32)
    pos_col = jnp.transpose(jnp.minimum(pos_row, float(k - 1)))
    pos_ref[...] = pos_col.astype(jnp.int32)
    msk_ref[...] = jnp.transpose(mask_row)


def _pool(x, p):
    K = x.shape[0]
    k = K // 2
    return pl.pallas_call(
        functools.partial(_pool_kernel, K=K, k=k),
        out_shape=[jax.ShapeDtypeStruct((K, C), jnp.float32),
                   jax.ShapeDtypeStruct((k, 1), jnp.int32),
                   jax.ShapeDtypeStruct((K, 1), jnp.int32),
                   jax.ShapeDtypeStruct((K, 1), jnp.float32)],
    )(x, p)


_BT = 512


def _tr_kernel(x_ref, o_ref):
    o_ref[...] = jnp.transpose(x_ref[...])


def _transpose(x):
    R, Q = x.shape
    return pl.pallas_call(
        _tr_kernel,
        grid=(R // _BT, Q // _BT),
        in_specs=[pl.BlockSpec((_BT, _BT), lambda i, j: (i, j))],
        out_specs=pl.BlockSpec((_BT, _BT), lambda i, j: (j, i)),
        out_shape=jax.ShapeDtypeStruct((Q, R), jnp.float32),
    )(x)


def _xw_kernel(x_ref, w_ref, b_ref, o_ref):
    o_ref[...] = (jnp.dot(x_ref[...], w_ref[...],
                          preferred_element_type=jnp.float32) + b_ref[...])


def _xw(x, w, b):
    M = x.shape[0]
    bm = min(M, 1024)
    return pl.pallas_call(
        _xw_kernel,
        grid=(M // bm,),
        in_specs=[
            pl.BlockSpec((bm, C), lambda i: (i, 0)),
            pl.BlockSpec((C, C), lambda i: (0, 0)),
            pl.BlockSpec((1, C), lambda i: (0, 0)),
        ],
        out_specs=pl.BlockSpec((bm, C), lambda i: (i, 0)),
        out_shape=jax.ShapeDtypeStruct((M, C), jnp.float32),
    )(x, w, b)


def _xw_sum_kernel(xa_ref, xb_ref, m_ref, w_ref, b_ref, o_ref):
    xs = xa_ref[...] + xb_ref[...] * m_ref[...]
    o_ref[...] = (jnp.dot(xs, w_ref[...],
                          preferred_element_type=jnp.float32) + b_ref[...])


def _xw_sum(xa, xb, m_col, w, b):
    M = xa.shape[0]
    bm = min(M, 1024)
    return pl.pallas_call(
        _xw_sum_kernel,
        grid=(M // bm,),
        in_specs=[
            pl.BlockSpec((bm, C), lambda i: (i, 0)),
            pl.BlockSpec((bm, C), lambda i: (i, 0)),
            pl.BlockSpec((bm, 1), lambda i: (i, 0)),
            pl.BlockSpec((C, C), lambda i: (0, 0)),
            pl.BlockSpec((1, C), lambda i: (0, 0)),
        ],
        out_specs=pl.BlockSpec((bm, C), lambda i: (i, 0)),
        out_shape=jax.ShapeDtypeStruct((M, C), jnp.float32),
    )(xa, xb, m_col, w, b)


def _hmm_kernel(h_ref, t_ref, o_ref):
    o_ref[...] = jnp.maximum(
        jnp.dot(h_ref[...], t_ref[...], preferred_element_type=jnp.float32), 0.0)


def _hmm_t_kernel(h_ref, t_ref, o_ref):
    acc = lax.dot_general(h_ref[...], t_ref[...], (((0,), (0,)), ((), ())),
                          preferred_element_type=jnp.float32)
    o_ref[...] = jnp.maximum(acc, 0.0)


def _hconv(S, t, transposed):
    M = t.shape[0]
    bm = min(M, 1024)
    if transposed:
        return pl.pallas_call(
            _hmm_t_kernel,
            grid=(M // bm,),
            in_specs=[
                pl.BlockSpec((M, bm), lambda i: (0, i)),
                pl.BlockSpec((M, C), lambda i: (0, 0)),
            ],
            out_specs=pl.BlockSpec((bm, C), lambda i: (i, 0)),
            out_shape=jax.ShapeDtypeStruct((M, C), jnp.float32),
        )(S, t)
    return pl.pallas_call(
        _hmm_kernel,
        grid=(M // bm,),
        in_specs=[
            pl.BlockSpec((bm, M), lambda i: (i, 0)),
            pl.BlockSpec((M, C), lambda i: (0, 0)),
        ],
        out_specs=pl.BlockSpec((bm, C), lambda i: (i, 0)),
        out_shape=jax.ShapeDtypeStruct((M, C), jnp.float32),
    )(S, t)


def _gather_rows(table, idx_2d):
    return _sc_gather(table, idx_2d.reshape(-1))


def kernel(feat, H, pool_w0, pool_w1, pool_w2, Wd0, bd0, Wd1, bd1, Wd2, bd2,
           Wu0, bu0, Wu1, bu1, Wu2, bu2):
    pool_ws = [pool_w0, pool_w1, pool_w2]
    Wds = [Wd0, Wd1, Wd2]
    bds = [bd0.reshape(1, C), bd1.reshape(1, C), bd2.reshape(1, C)]
    Wus = [Wu0, Wu1, Wu2]
    bus = [bu0.reshape(1, C), bu1.reshape(1, C), bu2.reshape(1, C)]

    x = feat
    S = H
    S_T = False
    xsaved = [feat]
    saved = [(H, False)]
    poss, msks = [], []
    for i in range(DEPTH):
        xs, perm, pos, msk = _pool(x, pool_ws[i])
        poss.append(pos)
        msks.append(msk)
        A = _gather_rows(S, perm)
        At = _transpose(A)
        S = _gather_rows(At, perm)
        S_T = not S_T
        xg = _gather_rows(xs, perm)
        t = _xw(xg, Wds[i], bds[i])
        x = _hconv(S, t, S_T)
        if i < DEPTH - 1:
            xsaved.append(x)
            saved.append((S, S_T))
    for i in range(DEPTH):
        j = DEPTH - i - 1
        Sj, Sj_T = saved[j]
        up = _gather_rows(x, poss[j])
        t = _xw_sum(xsaved[j], up, msks[j], Wus[i], bus[i])
        x = _hconv(Sj, t, Sj_T)
    return x
